# Initial kernel scaffold; baseline (speedup 1.0000x reference)
import jax
import jax.numpy as jnp
from jax.experimental import pallas as pl


def kernel(x, encoding_weight):
    raise NotImplementedError("write your pallas kernel here")


# speedup vs baseline: 2.1947x; 2.1947x over previous
import jax
import jax.numpy as jnp
from jax.experimental import pallas as pl


def _add_body(x_ref, w_ref, o_ref):
    o_ref[...] = x_ref[...] + w_ref[...]


def kernel(x, encoding_weight):
    B, T, D = x.shape
    TD = T * D
    x2 = x.reshape(B, TD)
    w2 = encoding_weight.reshape(1, TD)
    BB = 512
    out = pl.pallas_call(
        _add_body,
        grid=(B // BB,),
        in_specs=[
            pl.BlockSpec((BB, TD), lambda i: (i, 0)),
            pl.BlockSpec((1, TD), lambda i: (0, 0)),
        ],
        out_specs=pl.BlockSpec((BB, TD), lambda i: (i, 0)),
        out_shape=jax.ShapeDtypeStruct((B, TD), x.dtype),
    )(x2, w2)
    return out.reshape(B, T, D)


# speedup vs baseline: 2.2073x; 1.0058x over previous
import jax
import jax.numpy as jnp
from jax.experimental import pallas as pl


def _add_body(x_ref, w_ref, o_ref):
    o_ref[...] = x_ref[...] + w_ref[...]


def kernel(x, encoding_weight):
    B, T, D = x.shape
    TD = T * D
    x2 = x.reshape(B, TD)
    w2 = encoding_weight.reshape(1, TD)
    BB = 1024
    out = pl.pallas_call(
        _add_body,
        grid=(B // BB,),
        in_specs=[
            pl.BlockSpec((BB, TD), lambda i: (i, 0)),
            pl.BlockSpec((1, TD), lambda i: (0, 0)),
        ],
        out_specs=pl.BlockSpec((BB, TD), lambda i: (i, 0)),
        out_shape=jax.ShapeDtypeStruct((B, TD), x.dtype),
    )(x2, w2)
    return out.reshape(B, T, D)


# speedup vs baseline: 3.2501x; 1.4725x over previous
import jax
import jax.numpy as jnp
from jax.experimental import pallas as pl


def _add_body(x_ref, w_ref, o_ref):
    o_ref[...] = x_ref[...] + w_ref[...]


def kernel(x, encoding_weight):
    B, T, D = x.shape
    BB = 512
    return pl.pallas_call(
        _add_body,
        grid=(B // BB,),
        in_specs=[
            pl.BlockSpec((BB, T, D), lambda i: (i, 0, 0)),
            pl.BlockSpec((1, T, D), lambda i: (0, 0, 0)),
        ],
        out_specs=pl.BlockSpec((BB, T, D), lambda i: (i, 0, 0)),
        out_shape=jax.ShapeDtypeStruct((B, T, D), x.dtype),
    )(x, encoding_weight[None])


# speedup vs baseline: 3.4154x; 1.0508x over previous
import jax
import jax.numpy as jnp
from jax import lax
from jax.experimental import pallas as pl
from jax.experimental.pallas import tpu as pltpu

_CB = 256
_NBUF = 4


def _body(x_hbm, w_vmem, o_hbm, bufs, obufs, in_sems, out_sems):
    nchunk = x_hbm.shape[0] // _CB

    def in_copy(g, slot):
        return pltpu.make_async_copy(
            x_hbm.at[pl.ds(g * _CB, _CB)], bufs.at[slot], in_sems.at[slot])

    def out_copy(g, slot):
        return pltpu.make_async_copy(
            obufs.at[slot], o_hbm.at[pl.ds(g * _CB, _CB)], out_sems.at[slot])

    for s in range(_NBUF):
        in_copy(s, s).start()

    w = w_vmem[...]

    def step(g, carry):
        slot = lax.rem(g, _NBUF)
        in_copy(g, slot).wait()

        @pl.when(g >= _NBUF)
        def _():
            out_copy(g - _NBUF, slot).wait()

        obufs[slot] = bufs[slot] + w
        out_copy(g, slot).start()

        @pl.when(g + _NBUF < nchunk)
        def _():
            in_copy(g + _NBUF, slot).start()

        return carry

    lax.fori_loop(0, nchunk, step, 0)

    for s in range(_NBUF):
        out_copy(nchunk - _NBUF + s, s).wait()


def kernel(x, encoding_weight):
    B, T, D = x.shape
    return pl.pallas_call(
        _body,
        in_specs=[
            pl.BlockSpec(memory_space=pl.ANY),
            pl.BlockSpec(memory_space=pltpu.VMEM),
        ],
        out_specs=pl.BlockSpec(memory_space=pl.ANY),
        out_shape=jax.ShapeDtypeStruct((B, T, D), x.dtype),
        scratch_shapes=[
            pltpu.VMEM((_NBUF, _CB, T, D), jnp.float32),
            pltpu.VMEM((_NBUF, _CB, T, D), jnp.float32),
            pltpu.SemaphoreType.DMA((_NBUF,)),
            pltpu.SemaphoreType.DMA((_NBUF,)),
        ],
    )(x, encoding_weight)
